# baseline (speedup 1.0000x reference)
import numpy as np
import jax
import jax.numpy as jnp
from jax.experimental import pallas as pl
from jax.experimental.pallas import tpu as pltpu

_pc = pl.pallas_call

_BANK = 100000
_CHUNK = 10000
_NCHUNK = 10
_NQ = 1024
_DIM = 32
_K = 9
_NCAND = 16
_CHUNK2 = 2048
_NCHUNK2 = 49
_PAD2 = _CHUNK2 * _NCHUNK2
_TR = 784
_TL = 128


def _gauss_upsample_matrix():
    ks = 33
    sigma = 4.0
    xs = np.arange(ks, dtype=np.float64) - (ks - 1) * 0.5
    g = np.exp(-(xs ** 2) / (2.0 * sigma * sigma))
    g = g / g.sum()
    w = np.zeros((224, 32), dtype=np.float64)
    for i in range(224):
        for t in range(ks):
            r = i + t - 16
            if r < 0:
                r = -r
            elif r > 223:
                r = 446 - r
            w[i, r // 7] += g[t]
    return w.astype(np.float32)


_W_NP = _gauss_upsample_matrix()


def _min_d2_kernel(em2t_ref, et_ref, m_ref, o_ref):
    i = pl.program_id(0)
    m32 = m_ref[...].astype(jnp.float32)
    b2 = jnp.sum(m32 * m32, axis=1, keepdims=True)
    g = jax.lax.dot_general(m_ref[...], em2t_ref[...], (((1,), (0,)), ((), ())),
                            preferred_element_type=jnp.float32)
    d2 = b2 + g
    cmin = jnp.min(d2, axis=0, keepdims=True)
    acc = jnp.where(i == 0, cmin, jnp.minimum(o_ref[...], cmin))
    et = et_ref[...]
    a2 = jnp.sum(et * et, axis=0, keepdims=True)
    scores = jnp.sqrt(jnp.maximum(acc + a2, 0.0))
    o_ref[...] = jnp.where(i == _NCHUNK - 1, scores, acc)


def _cand_d2_kernel(ea_ref, m_ref, o_ref, s_ref, acc_ref):
    i = pl.program_id(0)
    m = m_ref[...]
    cat = jnp.concatenate([m, m * m], axis=1)
    d2 = jax.lax.dot_general(ea_ref[...], cat, (((1,), (1,)), ((), ())),
                             preferred_element_type=jnp.float32)
    o_ref[...] = d2
    col = jax.lax.broadcasted_iota(jnp.int32, d2.shape, 1) + i * _CHUNK2
    d2m = jnp.where(col < _BANK, d2, jnp.inf)
    cmin = jnp.min(d2m, axis=1, keepdims=True)
    acc_ref[...] = jnp.where(i == 0, cmin,
                             jnp.minimum(acc_ref[...], cmin))

    @pl.when(i == _NCHUNK2 - 1)
    def _():
        s_ref[...] = jnp.broadcast_to(acc_ref[...], (_NCAND, 128))


def _top9_kernel(d_ref, ea_ref, o_ref):
    d = d_ref[...]
    n = (jax.lax.broadcasted_iota(jnp.int32, d.shape, 0) * _TL
         + jax.lax.broadcasted_iota(jnp.int32, d.shape, 1))
    d = jnp.where(n < _BANK, d, jnp.inf)
    lane = jax.lax.broadcasted_iota(jnp.int32, (1, 128), 1)
    vals = jnp.zeros((1, 128), jnp.float32)
    for j in range(_K):
        v = jnp.min(d)
        idx = jnp.min(jnp.where(d == v, n, jnp.int32(2147483647)))
        vals = jnp.where(lane == j, v, vals)
        d = jnp.where(n == idx, jnp.inf, d)
    ea = ea_ref[...]
    a2 = jnp.sum(ea[:, :_DIM] * ea[:, :_DIM], axis=1, keepdims=True) * 0.25
    conf = jnp.sqrt(jnp.maximum(vals + a2, 0.0))
    conf = jnp.where(lane < _K, conf, -jnp.inf)
    ec = jnp.exp(conf)
    weights = 1.0 - jnp.max(ec) / jnp.sum(ec)
    c0 = jnp.max(jnp.where(lane == 0, conf, -jnp.inf))
    o_ref[...] = jnp.broadcast_to(weights * c0, (1, 128))


def _blur_kernel(w_ref, wt_ref, x_ref, o_ref):
    t = jax.lax.dot_general(w_ref[...], x_ref[...], (((1,), (0,)), ((), ())),
                            preferred_element_type=jnp.float32)
    o_ref[...] = jax.lax.dot_general(t, wt_ref[...], (((1,), (0,)), ((), ())),
                                     preferred_element_type=jnp.float32)


def kernel(embedding, memory_bank):
    e = embedding.astype(jnp.float32)
    m = memory_bank.astype(jnp.float32)
    m_bf = m.astype(jnp.bfloat16)
    et = e.T
    em2t_bf = (-2.0 * et).astype(jnp.bfloat16)

    minref = _pc(
        _min_d2_kernel,
        grid=(_NCHUNK,),
        in_specs=[
            pl.BlockSpec((_DIM, _NQ), lambda i: (0, 0)),
            pl.BlockSpec((_DIM, _NQ), lambda i: (0, 0)),
            pl.BlockSpec((_CHUNK, _DIM), lambda i: (i, 0)),
        ],
        out_specs=pl.BlockSpec((1, _NQ), lambda i: (0, 0)),
        out_shape=jax.ShapeDtypeStruct((1, _NQ), jnp.float32),
    )(em2t_bf, et, m_bf)
    scores = minref[0]

    w = jnp.asarray(_W_NP)
    amap = _pc(
        _blur_kernel,
        out_shape=jax.ShapeDtypeStruct((224, 224), jnp.float32),
    )(w, w.T, scores.reshape(32, 32))
    amap = amap.reshape(1, 1, 224, 224)

    _, cand_idx = jax.lax.top_k(scores, _NCAND)
    e_cand = e[cand_idx]
    ea_cand = jnp.concatenate(
        [-2.0 * e_cand, jnp.ones((_NCAND, _DIM), jnp.float32)], axis=1)
    d2all, smin = _pc(
        _cand_d2_kernel,
        grid=(_NCHUNK2,),
        in_specs=[
            pl.BlockSpec((_NCAND, 2 * _DIM), lambda i: (0, 0)),
            pl.BlockSpec((_CHUNK2, _DIM), lambda i: (i, 0)),
        ],
        out_specs=[
            pl.BlockSpec((_NCAND, _CHUNK2), lambda i: (0, i)),
            pl.BlockSpec((_NCAND, 128), lambda i: (0, 0)),
        ],
        out_shape=[
            jax.ShapeDtypeStruct((_NCAND, _PAD2), jnp.float32),
            jax.ShapeDtypeStruct((_NCAND, 128), jnp.float32),
        ],
        scratch_shapes=[pltpu.VMEM((_NCAND, 1), jnp.float32)],
    )(ea_cand, m)
    a2c = jnp.sum(e_cand * e_cand, axis=1)
    exact_c = jnp.sqrt(jnp.maximum(smin[:, 0] + a2c, 0.0))
    best = jnp.argmax(exact_c)

    d2best = jax.lax.dynamic_slice(d2all, (best, jnp.int32(0)), (1, _PAD2))
    ea_best = jax.lax.dynamic_slice(ea_cand, (best, jnp.int32(0)),
                                    (1, 2 * _DIM))
    out = _pc(
        _top9_kernel,
        out_shape=jax.ShapeDtypeStruct((1, 128), jnp.float32),
    )(d2best.reshape(_TR, _TL), ea_best)
    anomaly_score = out[0, 0]
    return (amap, anomaly_score)


# speedup vs baseline: 1.2075x; 1.2075x over previous
import numpy as np
import jax
import jax.numpy as jnp
from jax.experimental import pallas as pl

_pc = pl.pallas_call

_BANK = 100000
_CHUNK = 2000
_NCHUNK = 50
_NQ = 1024
_DIM = 32
_K = 9
_CHUNK2 = 2048
_NCHUNK2 = 49
_PAD2 = _CHUNK2 * _NCHUNK2
_TR = 784
_TL = 128


def _gauss_upsample_matrix():
    ks = 33
    sigma = 4.0
    xs = np.arange(ks, dtype=np.float64) - (ks - 1) * 0.5
    g = np.exp(-(xs ** 2) / (2.0 * sigma * sigma))
    g = g / g.sum()
    w = np.zeros((224, 32), dtype=np.float64)
    for i in range(224):
        for t in range(ks):
            r = i + t - 16
            if r < 0:
                r = -r
            elif r > 223:
                r = 446 - r
            w[i, r // 7] += g[t]
    return w.astype(np.float32)


_W_NP = _gauss_upsample_matrix()


def _min_d2_kernel(em2_ref, et_ref, m_ref, o_ref):
    i = pl.program_id(0)
    m = m_ref[...]
    b2 = jnp.sum(m * m, axis=1, keepdims=True)
    g = jax.lax.dot_general(m, em2_ref[...], (((1,), (1,)), ((), ())),
                            preferred_element_type=jnp.float32)
    d2 = b2 + g
    cmin = jnp.min(d2, axis=0, keepdims=True)
    acc = jnp.where(i == 0, cmin, jnp.minimum(o_ref[...], cmin))
    et = et_ref[...]
    a2 = jnp.sum(et * et, axis=0, keepdims=True)
    scores = jnp.sqrt(jnp.maximum(acc + a2, 0.0))
    o_ref[...] = jnp.where(i == _NCHUNK - 1, scores, acc)


def _row_d2_kernel(ea_ref, m_ref, o_ref):
    m = m_ref[...]
    cat = jnp.concatenate([m, m * m], axis=1)
    o_ref[...] = jax.lax.dot_general(
        ea_ref[...], cat, (((1,), (1,)), ((), ())),
        preferred_element_type=jnp.float32)


def _top9_kernel(d_ref, ea_ref, o_ref):
    d = d_ref[...]
    n = (jax.lax.broadcasted_iota(jnp.int32, d.shape, 0) * _TL
         + jax.lax.broadcasted_iota(jnp.int32, d.shape, 1))
    d = jnp.where(n < _BANK, d, jnp.inf)
    lane = jax.lax.broadcasted_iota(jnp.int32, (1, 128), 1)
    vals = jnp.zeros((1, 128), jnp.float32)
    for j in range(_K):
        v = jnp.min(d)
        idx = jnp.min(jnp.where(d == v, n, jnp.int32(2147483647)))
        vals = jnp.where(lane == j, v, vals)
        d = jnp.where(n == idx, jnp.inf, d)
    ea = ea_ref[...]
    a2 = jnp.sum(ea[:, :_DIM] * ea[:, :_DIM], axis=1, keepdims=True) * 0.25
    conf = jnp.sqrt(jnp.maximum(vals + a2, 0.0))
    conf = jnp.where(lane < _K, conf, -jnp.inf)
    ec = jnp.exp(conf)
    weights = 1.0 - jnp.max(ec) / jnp.sum(ec)
    c0 = jnp.max(jnp.where(lane == 0, conf, -jnp.inf))
    o_ref[...] = jnp.broadcast_to(weights * c0, (1, 128))


def _blur_kernel(w_ref, wt_ref, x_ref, o_ref):
    t = jax.lax.dot_general(w_ref[...], x_ref[...], (((1,), (0,)), ((), ())),
                            preferred_element_type=jnp.float32)
    o_ref[...] = jax.lax.dot_general(t, wt_ref[...], (((1,), (0,)), ((), ())),
                                     preferred_element_type=jnp.float32)


def kernel(embedding, memory_bank):
    e = embedding.astype(jnp.float32)
    m = memory_bank.astype(jnp.float32)
    em2 = -2.0 * e
    et = e.T

    minref = _pc(
        _min_d2_kernel,
        grid=(_NCHUNK,),
        in_specs=[
            pl.BlockSpec((_NQ, _DIM), lambda i: (0, 0)),
            pl.BlockSpec((_DIM, _NQ), lambda i: (0, 0)),
            pl.BlockSpec((_CHUNK, _DIM), lambda i: (i, 0)),
        ],
        out_specs=pl.BlockSpec((1, _NQ), lambda i: (0, 0)),
        out_shape=jax.ShapeDtypeStruct((1, _NQ), jnp.float32),
    )(em2, et, m)
    scores = minref[0]

    w = jnp.asarray(_W_NP)
    amap = _pc(
        _blur_kernel,
        out_shape=jax.ShapeDtypeStruct((224, 224), jnp.float32),
    )(w, w.T, scores.reshape(32, 32))
    amap = amap.reshape(1, 1, 224, 224)

    max_idx = jnp.argmax(scores)
    e_row = jax.lax.dynamic_slice(e, (max_idx, jnp.int32(0)), (1, _DIM))
    ea_row = jnp.concatenate([-2.0 * e_row, jnp.ones((1, _DIM), jnp.float32)],
                             axis=1)
    d2row = _pc(
        _row_d2_kernel,
        grid=(_NCHUNK2,),
        in_specs=[
            pl.BlockSpec((1, 2 * _DIM), lambda i: (0, 0)),
            pl.BlockSpec((_CHUNK2, _DIM), lambda i: (i, 0)),
        ],
        out_specs=pl.BlockSpec((1, _CHUNK2), lambda i: (0, i)),
        out_shape=jax.ShapeDtypeStruct((1, _PAD2), jnp.float32),
    )(ea_row, m)
    out = _pc(
        _top9_kernel,
        out_shape=jax.ShapeDtypeStruct((1, 128), jnp.float32),
    )(d2row.reshape(_TR, _TL), ea_row)
    anomaly_score = out[0, 0]
    return (amap, anomaly_score)
